# baseline (speedup 1.0000x reference)
import jax
import jax.numpy as jnp
from jax.experimental import pallas as pl
from jax.experimental.pallas import tpu as pltpu

N = 100000
LATENT = 32
LB = 6400
GRID = (N + LB - 1) // LB


def kernel(z, att_w1, att_b1, att_w2, att_b2,
           mlp_w1, mlp_b1, mlp_w2, mlp_b2, mlp_w3, mlp_b3):
    zt = z.T
    w1t = att_w1.T
    b1c = att_b1.reshape(LATENT, 1)
    w2r = jnp.tile(att_w2.T, (8, 1))
    mw1t = mlp_w1.T
    mb1c = mlp_b1.reshape(128, 1)
    mw2t = mlp_w2.T
    mb2c = mlp_b2.reshape(64, 1)
    mw3t = mlp_w3.T
    small = lambda shape: pl.BlockSpec(shape, lambda i: tuple(0 for _ in shape))

    def body(z_ref, w1_ref, b1_ref, w2_ref, mw1_ref, mb1_ref,
             mw2_ref, mb2_ref, mw3_ref, mb3_ref, out_ref, av_ref, ae_ref):
        i = pl.program_id(0)
        nsteps = pl.num_programs(0)

        @pl.when(i == 0)
        def _():
            av_ref[...] = jnp.zeros_like(av_ref)
            ae_ref[...] = jnp.zeros_like(ae_ref)

        zb = z_ref[...]
        h = jnp.tanh(w1_ref[...] @ zb + b1_ref[...])
        e8 = jnp.exp(w2_ref[...] @ h)

        def accumulate(p, e8v):
            av = av_ref[...]
            ae = ae_ref[...]
            for c in range(LB // 128):
                av += p[:, 128 * c:128 * (c + 1)]
                ae += e8v[:, 128 * c:128 * (c + 1)]
            av_ref[...] = av
            ae_ref[...] = ae

        @pl.when(i < nsteps - 1)
        def _():
            accumulate(zb * e8[0:1, :], e8)

        @pl.when(i == nsteps - 1)
        def _():
            valid = N - (nsteps - 1) * LB
            lane8 = jax.lax.broadcasted_iota(jnp.int32, (8, LB), 1)
            lane32 = jax.lax.broadcasted_iota(jnp.int32, (LATENT, LB), 1)
            e8m = jnp.where(lane8 < valid, e8, 0.0)
            pm = jnp.where(lane32 < valid, zb * e8[0:1, :], 0.0)
            accumulate(pm, e8m)

            s = jnp.sum(ae_ref[...]) * 0.125
            vz = jnp.sum(av_ref[...], axis=1, keepdims=True)
            g = vz / (s + 1e-8)
            x = jnp.maximum(mw1_ref[...] @ g + mb1_ref[...], 0.0)
            x = jnp.maximum(mw2_ref[...] @ x + mb2_ref[...], 0.0)
            y = mw3_ref[...] @ x + mb3_ref[...]
            out_ref[...] = jax.nn.sigmoid(y)

    out = pl.pallas_call(
        body,
        grid=(GRID,),
        in_specs=[
            pl.BlockSpec((LATENT, LB), lambda i: (0, i)),
            small((LATENT, LATENT)),
            small((LATENT, 1)),
            small((8, LATENT)),
            small((128, LATENT)),
            small((128, 1)),
            small((64, 128)),
            small((64, 1)),
            small((1, 64)),
            small((1, 1)),
        ],
        out_specs=pl.BlockSpec((1, 1), lambda i: (0, 0)),
        out_shape=jax.ShapeDtypeStruct((1, 1), jnp.float32),
        scratch_shapes=[
            pltpu.VMEM((LATENT, 128), jnp.float32),
            pltpu.VMEM((8, 128), jnp.float32),
        ],
        compiler_params=pltpu.CompilerParams(
            dimension_semantics=("arbitrary",),
        ),
    )(
        zt, w1t, b1c, w2r,
        mw1t, mb1c, mw2t, mb2c, mw3t, mlp_b3.reshape(1, 1),
    )
    return out.reshape(-1)


# speedup vs baseline: 1.3659x; 1.3659x over previous
import jax
import jax.numpy as jnp
from jax.experimental import pallas as pl
from jax.experimental.pallas import tpu as pltpu

N = 100000
LATENT = 32
LB = 25600
GRID = (N + LB - 1) // LB


def kernel(z, att_w1, att_b1, att_w2, att_b2,
           mlp_w1, mlp_b1, mlp_w2, mlp_b2, mlp_w3, mlp_b3):
    zt = z.T
    w1t = att_w1.T
    b1c = att_b1.reshape(LATENT, 1)
    w2r = jnp.tile(att_w2.T, (8, 1))
    mw1t = mlp_w1.T
    mb1c = mlp_b1.reshape(128, 1)
    mw2t = mlp_w2.T
    mb2c = mlp_b2.reshape(64, 1)
    mw3t = mlp_w3.T
    small = lambda shape: pl.BlockSpec(shape, lambda i: tuple(0 for _ in shape))

    def body(z_ref, w1_ref, b1_ref, w2_ref, mw1_ref, mb1_ref,
             mw2_ref, mb2_ref, mw3_ref, mb3_ref, out_ref, av_ref, ae_ref):
        i = pl.program_id(0)
        nsteps = pl.num_programs(0)

        @pl.when(i == 0)
        def _():
            av_ref[...] = jnp.zeros_like(av_ref)
            ae_ref[...] = jnp.zeros_like(ae_ref)

        zb = z_ref[...]
        h = jnp.tanh(w1_ref[...] @ zb + b1_ref[...])
        e8 = jnp.exp(w2_ref[...] @ h)

        def accumulate(p, e8v):
            av = av_ref[...]
            ae = ae_ref[...]
            for c in range(LB // 128):
                av += p[:, 128 * c:128 * (c + 1)]
                ae += e8v[:, 128 * c:128 * (c + 1)]
            av_ref[...] = av
            ae_ref[...] = ae

        @pl.when(i < nsteps - 1)
        def _():
            accumulate(zb * e8[0:1, :], e8)

        @pl.when(i == nsteps - 1)
        def _():
            valid = N - (nsteps - 1) * LB
            lane8 = jax.lax.broadcasted_iota(jnp.int32, (8, LB), 1)
            lane32 = jax.lax.broadcasted_iota(jnp.int32, (LATENT, LB), 1)
            e8m = jnp.where(lane8 < valid, e8, 0.0)
            pm = jnp.where(lane32 < valid, zb * e8[0:1, :], 0.0)
            accumulate(pm, e8m)

            s = jnp.sum(ae_ref[...]) * 0.125
            vz = jnp.sum(av_ref[...], axis=1, keepdims=True)
            g = vz / (s + 1e-8)
            x = jnp.maximum(mw1_ref[...] @ g + mb1_ref[...], 0.0)
            x = jnp.maximum(mw2_ref[...] @ x + mb2_ref[...], 0.0)
            y = mw3_ref[...] @ x + mb3_ref[...]
            out_ref[...] = jax.nn.sigmoid(y)

    out = pl.pallas_call(
        body,
        grid=(GRID,),
        in_specs=[
            pl.BlockSpec((LATENT, LB), lambda i: (0, i)),
            small((LATENT, LATENT)),
            small((LATENT, 1)),
            small((8, LATENT)),
            small((128, LATENT)),
            small((128, 1)),
            small((64, 128)),
            small((64, 1)),
            small((1, 64)),
            small((1, 1)),
        ],
        out_specs=pl.BlockSpec((1, 1), lambda i: (0, 0)),
        out_shape=jax.ShapeDtypeStruct((1, 1), jnp.float32),
        scratch_shapes=[
            pltpu.VMEM((LATENT, 128), jnp.float32),
            pltpu.VMEM((8, 128), jnp.float32),
        ],
        compiler_params=pltpu.CompilerParams(
            dimension_semantics=("arbitrary",),
        ),
    )(
        zt, w1t, b1c, w2r,
        mw1t, mb1c, mw2t, mb2c, mw3t, mlp_b3.reshape(1, 1),
    )
    return out.reshape(-1)


# speedup vs baseline: 1.3702x; 1.0032x over previous
import jax
import jax.numpy as jnp
from jax.experimental import pallas as pl
from jax.experimental.pallas import tpu as pltpu

N = 100000
LATENT = 32
LB = 51200
GRID = (N + LB - 1) // LB


def kernel(z, att_w1, att_b1, att_w2, att_b2,
           mlp_w1, mlp_b1, mlp_w2, mlp_b2, mlp_w3, mlp_b3):
    zt = z.T
    w1t = att_w1.T
    b1c = att_b1.reshape(LATENT, 1)
    w2r = jnp.tile(att_w2.T, (8, 1))
    mw1t = mlp_w1.T
    mb1c = mlp_b1.reshape(128, 1)
    mw2t = mlp_w2.T
    mb2c = mlp_b2.reshape(64, 1)
    mw3t = mlp_w3.T
    small = lambda shape: pl.BlockSpec(shape, lambda i: tuple(0 for _ in shape))

    def body(z_ref, w1_ref, b1_ref, w2_ref, mw1_ref, mb1_ref,
             mw2_ref, mb2_ref, mw3_ref, mb3_ref, out_ref, av_ref, ae_ref):
        i = pl.program_id(0)
        nsteps = pl.num_programs(0)

        @pl.when(i == 0)
        def _():
            av_ref[...] = jnp.zeros_like(av_ref)
            ae_ref[...] = jnp.zeros_like(ae_ref)

        zb = z_ref[...]
        h = jnp.tanh(w1_ref[...] @ zb + b1_ref[...])
        e8 = jnp.exp(w2_ref[...] @ h)

        def accumulate(p, e8v):
            av = av_ref[...]
            ae = ae_ref[...]
            for c in range(LB // 128):
                av += p[:, 128 * c:128 * (c + 1)]
                ae += e8v[:, 128 * c:128 * (c + 1)]
            av_ref[...] = av
            ae_ref[...] = ae

        @pl.when(i < nsteps - 1)
        def _():
            accumulate(zb * e8[0:1, :], e8)

        @pl.when(i == nsteps - 1)
        def _():
            valid = N - (nsteps - 1) * LB
            lane8 = jax.lax.broadcasted_iota(jnp.int32, (8, LB), 1)
            lane32 = jax.lax.broadcasted_iota(jnp.int32, (LATENT, LB), 1)
            e8m = jnp.where(lane8 < valid, e8, 0.0)
            pm = jnp.where(lane32 < valid, zb * e8[0:1, :], 0.0)
            accumulate(pm, e8m)

            s = jnp.sum(ae_ref[...]) * 0.125
            vz = jnp.sum(av_ref[...], axis=1, keepdims=True)
            g = vz / (s + 1e-8)
            x = jnp.maximum(mw1_ref[...] @ g + mb1_ref[...], 0.0)
            x = jnp.maximum(mw2_ref[...] @ x + mb2_ref[...], 0.0)
            y = mw3_ref[...] @ x + mb3_ref[...]
            out_ref[...] = jax.nn.sigmoid(y)

    out = pl.pallas_call(
        body,
        grid=(GRID,),
        in_specs=[
            pl.BlockSpec((LATENT, LB), lambda i: (0, i)),
            small((LATENT, LATENT)),
            small((LATENT, 1)),
            small((8, LATENT)),
            small((128, LATENT)),
            small((128, 1)),
            small((64, 128)),
            small((64, 1)),
            small((1, 64)),
            small((1, 1)),
        ],
        out_specs=pl.BlockSpec((1, 1), lambda i: (0, 0)),
        out_shape=jax.ShapeDtypeStruct((1, 1), jnp.float32),
        scratch_shapes=[
            pltpu.VMEM((LATENT, 128), jnp.float32),
            pltpu.VMEM((8, 128), jnp.float32),
        ],
        compiler_params=pltpu.CompilerParams(
            dimension_semantics=("arbitrary",),
        ),
    )(
        zt, w1t, b1c, w2r,
        mw1t, mb1c, mw2t, mb2c, mw3t, mlp_b3.reshape(1, 1),
    )
    return out.reshape(-1)
